# baseline (speedup 1.0000x reference)
import functools

import jax
import jax.numpy as jnp
from jax import lax
from jax.experimental import pallas as pl
from jax.experimental.pallas import tpu as pltpu
from jax.experimental.pallas import tpu_sc as plsc

_NUM_EMB = 1000002
_D = 64
_B = 16384
_NW = 32
_BW = _B // _NW
_CHUNK = 128
_NCHUNK = _BW // _CHUNK


def _rsqrt16(x):
    i = lax.bitcast_convert_type(x, jnp.int32)
    i = jnp.int32(0x5F3759DF) - lax.shift_right_logical(i, 1)
    y = lax.bitcast_convert_type(i, jnp.float32)
    for _ in range(3):
        y = y * (jnp.float32(1.5) - jnp.float32(0.5) * x * y * y)
    return y


def _sc_body(table_hbm, nodes_hbm, out_hbm, idx_v, rows_v, buf_t, scat_idx,
             gsem, ssem):
    wid = lax.axis_index("s") * 2 + lax.axis_index("c")
    iota = lax.iota(jnp.int32, 16)

    pltpu.sync_copy(nodes_hbm.at[wid], idx_v)

    copies = [
        pltpu.make_async_copy(
            table_hbm.at[idx_v.at[j]],
            rows_v.at[pl.ds(j * _CHUNK, _CHUNK)],
            gsem,
        )
        for j in range(_NCHUNK)
    ]
    for c in copies:
        c.start()

    for k in range(_D // 16):
        scat_idx[pl.ds(k * 16, 16)] = (k * 16 + iota) * _NW + wid

    for c in copies:
        c.wait()

    perms = [lax.rem(iota + d, jnp.int32(_D)) for d in range(_D)]

    def block(iblk, _):
        row = iblk * 16 + iota
        acc = jnp.zeros((16,), jnp.float32)
        for d in range(_D):
            v = plsc.load_gather(rows_v, [row, perms[d]])
            acc = acc + v * v
        r = _rsqrt16(acc)
        for d in range(_D):
            v = plsc.load_gather(rows_v, [row, perms[d]])
            plsc.store_scatter(buf_t, [perms[d], row], v * r)
        return 0

    lax.fori_loop(0, _BW // 16, block, 0)

    copy = pltpu.make_async_copy(buf_t, out_hbm.at[scat_idx], ssem)
    copy.start()
    copy.wait()


@functools.partial(jax.jit, static_argnames=())
def _sc_call(table, nodes3):
    mesh = plsc.VectorSubcoreMesh(core_axis_name="c", subcore_axis_name="s")
    return pl.kernel(
        _sc_body,
        out_type=jax.ShapeDtypeStruct((_D * _NW, _BW), jnp.float32),
        mesh=mesh,
        compiler_params=pltpu.CompilerParams(
            needs_layout_passes=False, use_tc_tiling_on_sc=False
        ),
        scratch_types=[
            pltpu.VMEM((_NCHUNK, _CHUNK), jnp.int32),
            pltpu.VMEM((_BW, _D), jnp.float32),
            pltpu.VMEM((_D, _BW), jnp.float32),
            pltpu.VMEM((_D,), jnp.int32),
            pltpu.SemaphoreType.DMA,
            pltpu.SemaphoreType.DMA,
        ],
    )(table, nodes3)


def kernel(nodes, table):
    nodes3 = nodes.reshape(_NW, _NCHUNK, _CHUNK)
    out2d = _sc_call(table, nodes3)
    return out2d.reshape(_D, _B)
